# baseline (speedup 1.0000x reference)
import jax
import jax.numpy as jnp
from jax import lax
from jax.experimental import pallas as pl
from jax.experimental.pallas import tpu as pltpu
from jax.experimental.pallas import tpu_sc as plsc

_N = 10000
_E = 320000
_DIN, _DH, _DOUT = 128, 256, 128
_NC, _NS, _BK = 2, 16, 128
_HBK = 64
_TRASH = _N
_NPAD = 10240
_DEGPAD = _NPAD
_DEGSL = _DEGPAD // _NS
_RPT = _NPAD // _NS
_NB16 = 160
_NB32 = 80
_EPAD = _NS * _NB16 * _BK
_CH = 16
_NHALF = _N // 2
_HROWS = 5120
_HRPT = _HROWS // _NS
_LTRASH = _NHALF
_CAP = 11264
_CAPP = _CAP + 8
_CROWS = _CAP // _BK
_PCH = 8
_F32 = jnp.float32
_HIGH = lax.Precision.HIGHEST

_MESH = plsc.VectorSubcoreMesh(
    core_axis_name="c", subcore_axis_name="s",
    num_cores=_NC, num_subcores=_NS)


def _deg_body(dst_h, ones_h, zero_h, deg_h, dst_v, ones_v, acc, sem):
    del sem
    c = lax.axis_index("c")
    s = lax.axis_index("s")
    w = c * _NS + s
    pltpu.sync_copy(dst_h.at[w], dst_v)
    pltpu.sync_copy(ones_h, ones_v)
    z0 = s * _DEGSL
    pltpu.sync_copy(zero_h.at[pl.ds(z0, _DEGSL)], acc.at[pl.ds(z0, _DEGSL)])
    plsc.subcore_barrier()

    def body(j, carry):
        pltpu.sync_copy(ones_v, acc.at[dst_v.at[j]], add=True)
        return carry

    lax.fori_loop(0, _NB32, body, 0)
    plsc.subcore_barrier()
    pltpu.sync_copy(acc.at[pl.ds(z0, _DEGSL)], deg_h.at[c, pl.ds(z0, _DEGSL)])


_deg_call = pl.kernel(
    _deg_body,
    out_type=jax.ShapeDtypeStruct((_NC, _DEGPAD), _F32),
    mesh=_MESH,
    scratch_types=[
        pltpu.VMEM((_NB32, _BK), jnp.int32),
        pltpu.VMEM((_BK,), _F32),
        pltpu.VMEM_SHARED((_DEGPAD,), _F32),
        pltpu.SemaphoreType.DMA,
    ],
)


def _scat_body(src_h, dst_h, posa_h, posb_h, dlb_h,
               sa_h, da_h, sb_h, db_h, pos_v, val_v,
               p2_v, v2_v, padz_v, padt_v, s_acc, d_acc):
    c = lax.axis_index("c")
    s = lax.axis_index("s")
    two = jnp.full((16,), 2, jnp.int32)
    one = jnp.full((16,), 1, jnp.int32)

    def fillz(i, carry):
        padz_v[pl.ds(i * 16, 16)] = jnp.zeros((16,), jnp.int32)
        padt_v[pl.ds(i * 16, 16)] = jnp.full((16,), 2 * _LTRASH, jnp.int32)
        return carry

    lax.fori_loop(0, (2 * _CAPP) // 16 + 1, fillz, 0)
    r0 = s * _CAPP
    r2 = s * (2 * _CAPP)
    pltpu.sync_copy(padz_v.at[pl.ds(0, _CAPP)], s_acc.at[pl.ds(r0, _CAPP)])
    pltpu.sync_copy(padt_v.at[pl.ds(0, 2 * _CAPP)],
                    d_acc.at[pl.ds(r2, 2 * _CAPP)])
    plsc.subcore_barrier()

    def do_side(pos_h, dval_h):
        def chunk(k, carry):
            pltpu.sync_copy(pos_h.at[s, pl.ds(k * _CH, _CH)], pos_v)
            pltpu.sync_copy(src_h.at[s, pl.ds(k * _CH, _CH)], val_v)
            for j in range(_CH):
                pltpu.sync_copy(val_v.at[j], s_acc.at[pos_v.at[j]])
            pltpu.sync_copy(dval_h.at[s, pl.ds(k * _CH, _CH)], val_v)

            def drow(j, cc):
                for l in range(_BK // 16):
                    sl = pl.ds(l * 16, 16)
                    p2_v[sl] = pos_v[j, sl] * two
                    v2_v[sl] = val_v[j, sl] * two
                pltpu.sync_copy(v2_v, d_acc.at[p2_v])
                for l in range(_BK // 16):
                    sl = pl.ds(l * 16, 16)
                    p2_v[sl] = p2_v[sl] + one
                    v2_v[sl] = v2_v[sl] + one
                pltpu.sync_copy(v2_v, d_acc.at[p2_v])
                return cc

            lax.fori_loop(0, _CH, drow, carry)
            return carry

        lax.fori_loop(0, _NB16 // _CH, chunk, 0)

    @pl.when(c == 0)
    def _():
        do_side(posa_h, dst_h)

    @pl.when(c == 1)
    def _():
        do_side(posb_h, dlb_h)

    plsc.subcore_barrier()

    pltpu.sync_copy(s_acc.at[pl.ds(r0, _CAPP)], padz_v.at[pl.ds(0, _CAPP)])
    pltpu.sync_copy(d_acc.at[pl.ds(r2, 2 * _CAPP)],
                    padt_v.at[pl.ds(0, 2 * _CAPP)])

    @pl.when(c == 0)
    def _():
        pltpu.sync_copy(padz_v.at[pl.ds(0, _CAPP)], sa_h.at[pl.ds(r0, _CAPP)])
        pltpu.sync_copy(padt_v.at[pl.ds(0, 2 * _CAPP)],
                        da_h.at[pl.ds(r2, 2 * _CAPP)])

    @pl.when(c == 1)
    def _():
        pltpu.sync_copy(padz_v.at[pl.ds(0, _CAPP)], sb_h.at[pl.ds(r0, _CAPP)])
        pltpu.sync_copy(padt_v.at[pl.ds(0, 2 * _CAPP)],
                        db_h.at[pl.ds(r2, 2 * _CAPP)])


_scat_call = pl.kernel(
    _scat_body,
    out_type=(jax.ShapeDtypeStruct((_NS * _CAPP,), jnp.int32),
              jax.ShapeDtypeStruct((_NS * _CAPP * 2,), jnp.int32),
              jax.ShapeDtypeStruct((_NS * _CAPP,), jnp.int32),
              jax.ShapeDtypeStruct((_NS * _CAPP * 2,), jnp.int32)),
    mesh=_MESH,
    scratch_types=[
        pltpu.VMEM((_CH, _BK), jnp.int32),
        pltpu.VMEM((_CH, _BK), jnp.int32),
        pltpu.VMEM((_BK,), jnp.int32),
        pltpu.VMEM((_BK,), jnp.int32),
        pltpu.VMEM((_CAPP,), jnp.int32),
        pltpu.VMEM((2 * _CAPP + 16,), jnp.int32),
        pltpu.VMEM_SHARED((_NS * _CAPP,), jnp.int32),
        pltpu.VMEM_SHARED((_NS * _CAPP * 2,), jnp.int32),
    ],
)


def _spmm_part_body(gf_h, sa_h, da_h, sb_h, db_h, zero_h, agg_h,
                    src_v, dst_v, rows0, rows1, acc, sem0, sem1):
    c = lax.axis_index("c")
    s = lax.axis_index("s")
    r0 = s * (2 * _HRPT)
    pltpu.sync_copy(zero_h.at[pl.ds(r0, 2 * _HRPT)],
                    acc.at[pl.ds(r0, 2 * _HRPT)])
    plsc.subcore_barrier()

    bufs = (rows0, rows1)
    sems = (sem0, sem1)

    def side_loop(s_h, d_h):
        def fire(r, h, p):
            return pltpu.async_copy(
                gf_h.at[src_v.at[r, pl.ds(h * _HBK, _HBK)]],
                bufs[p].reshape(_HBK, 2, _DH // 2), sems[p])

        def outer(k, carry):
            pltpu.sync_copy(s_h.at[s, pl.ds(k * _PCH, _PCH)], src_v)
            pltpu.sync_copy(d_h.at[s, pl.ds(2 * k * _PCH, 2 * _PCH)], dst_v)
            cps = [None, None]
            cps[0] = fire(0, 0, 0)
            for b in range(2 * _PCH):
                p = b & 1
                if b + 1 < 2 * _PCH:
                    cps[1 - p] = fire((b + 1) // 2, (b + 1) % 2, 1 - p)
                cps[p].wait()
                pltpu.sync_copy(bufs[p], acc.at[dst_v.at[b]], add=True)
            return carry

        lax.fori_loop(0, _CROWS // _PCH, outer, 0)

    @pl.when(c == 0)
    def _():
        side_loop(sa_h, da_h)

    @pl.when(c == 1)
    def _():
        side_loop(sb_h, db_h)

    plsc.subcore_barrier()
    pltpu.sync_copy(acc.at[pl.ds(r0, 2 * _HRPT)],
                    agg_h.at[c, pl.ds(r0, 2 * _HRPT)])


_spmm_part_call = pl.kernel(
    _spmm_part_body,
    out_type=jax.ShapeDtypeStruct((_NC, 2 * _HROWS, _DH // 2), _F32),
    mesh=_MESH,
    scratch_types=[
        pltpu.VMEM((_PCH, _BK), jnp.int32),
        pltpu.VMEM((2 * _PCH, _BK), jnp.int32),
        pltpu.VMEM((_BK, _DH // 2), _F32),
        pltpu.VMEM((_BK, _DH // 2), _F32),
        pltpu.VMEM_SHARED((2 * _HROWS, _DH // 2), _F32),
        pltpu.SemaphoreType.DMA,
        pltpu.SemaphoreType.DMA,
    ],
)


def _spmm_edge_body(g_h, src_h, dst_h, p0_h, p1_h,
                    src_v, dst_v, rows0, rows1, acc, sem0, sem1):
    c = lax.axis_index("c")
    s = lax.axis_index("s")
    w = c * _NS + s
    r0 = s * _RPT
    pltpu.sync_copy(g_h.at[pl.ds(r0, _RPT)], acc.at[pl.ds(r0, _RPT)])
    plsc.subcore_barrier()

    bufs = (rows0, rows1)
    sems = (sem0, sem1)

    def outer(k, carry):
        pltpu.sync_copy(src_h.at[w, pl.ds(k * _CH, _CH)], src_v)
        pltpu.sync_copy(dst_h.at[w, pl.ds(k * _CH, _CH)], dst_v)
        cps = [None, None]
        cps[0] = pltpu.async_copy(g_h.at[src_v.at[0]], bufs[0], sems[0])
        for j in range(_CH):
            p = j & 1
            if j + 1 < _CH:
                cps[1 - p] = pltpu.async_copy(
                    g_h.at[src_v.at[j + 1]], bufs[1 - p], sems[1 - p])
            cps[p].wait()
            pltpu.sync_copy(bufs[p], acc.at[dst_v.at[j]], add=True)
        return carry

    lax.fori_loop(0, _NB32 // _CH, outer, 0)
    plsc.subcore_barrier()

    @pl.when(c == 0)
    def _():
        pltpu.sync_copy(acc.at[pl.ds(r0, _RPT)], p0_h.at[pl.ds(r0, _RPT)])

    @pl.when(c == 1)
    def _():
        pltpu.sync_copy(acc.at[pl.ds(r0, _RPT)], p1_h.at[pl.ds(r0, _RPT)])


_spmm_edge_call = pl.kernel(
    _spmm_edge_body,
    out_type=(jax.ShapeDtypeStruct((_NPAD, _DOUT), _F32),
              jax.ShapeDtypeStruct((_NPAD, _DOUT), _F32)),
    mesh=_MESH,
    scratch_types=[
        pltpu.VMEM((_CH, _BK), jnp.int32),
        pltpu.VMEM((_CH, _BK), jnp.int32),
        pltpu.VMEM((_BK, _DOUT), _F32),
        pltpu.VMEM((_BK, _DOUT), _F32),
        pltpu.VMEM_SHARED((_NPAD, _DOUT), _F32),
        pltpu.SemaphoreType.DMA,
        pltpu.SemaphoreType.DMA,
    ],
)


_BN = 1000
_BNO = 2000


def _half_map(i):
    half = jnp.where(i < _N // (2 * _BN), 0, 1)
    return (half, i - half * (_N // (2 * _BN)), 0)


def _half_map4(i):
    half = jnp.where(i < _N // (2 * _BN), 0, 1)
    return (half, i - half * (_N // (2 * _BN)), 0, 0)


def _pos_body(dst_ref, posa_ref, posb_ref, dlb_ref):
    t = pl.program_id(0)
    d = dst_ref[0]
    col = lax.broadcasted_iota(jnp.int32, (_BK, _BK), 0)
    row = lax.broadcasted_iota(jnp.int32, (_BK, _BK), 1)
    slt = jnp.where(col < row, 1.0, 0.0).astype(_F32)
    ri = lax.broadcasted_iota(jnp.int32, (_NB16, _NB16), 0)
    ci = lax.broadcasted_iota(jnp.int32, (_NB16, _NB16), 1)
    ltr = jnp.where(ci < ri, 1.0, 0.0).astype(_F32)
    base = (t * _CAPP).astype(_F32)
    for m, ref in (((d < _NHALF), posa_ref),
                   (jnp.logical_and(d >= _NHALF, d < _N), posb_ref)):
        mf = jnp.where(m, 1.0, 0.0).astype(_F32)
        w = jnp.dot(mf, slt, preferred_element_type=_F32, precision=_HIGH)
        rs = jnp.sum(mf, axis=1, keepdims=True)
        rb = jnp.dot(ltr, rs, preferred_element_type=_F32, precision=_HIGH)
        pos = jnp.where(m, w + rb, jnp.float32(_CAP))
        ref[0] = (pos + base).astype(jnp.int32)
    dlb_ref[0] = d - _NHALF


_pos_call = pl.pallas_call(
    _pos_body,
    grid=(_NS,),
    in_specs=[pl.BlockSpec((1, _NB16, _BK), lambda i: (i, 0, 0))],
    out_specs=(pl.BlockSpec((1, _NB16, _BK), lambda i: (i, 0, 0)),
               pl.BlockSpec((1, _NB16, _BK), lambda i: (i, 0, 0)),
               pl.BlockSpec((1, _NB16, _BK), lambda i: (i, 0, 0))),
    out_shape=(jax.ShapeDtypeStruct((_NS, _NB16, _BK), jnp.int32),
               jax.ShapeDtypeStruct((_NS, _NB16, _BK), jnp.int32),
               jax.ShapeDtypeStruct((_NS, _NB16, _BK), jnp.int32)),
)


def _b1_body(x_ref, w1_ref, degt_ref, gf_ref, dinv_ref):
    deg = degt_ref[:, 0:1] + degt_ref[:, 1:2] + 1.0
    dinv = lax.rsqrt(deg)
    h = jnp.dot(x_ref[...], w1_ref[...],
                preferred_element_type=_F32, precision=_HIGH)
    g = h * dinv
    gf_ref[:, 0, :] = g[:, :_DH // 2]
    gf_ref[:, 1, :] = g[:, _DH // 2:]
    dinv_ref[...] = dinv


_b1_call = pl.pallas_call(
    _b1_body,
    grid=(_N // _BN,),
    in_specs=[
        pl.BlockSpec((_BN, _DIN), lambda i: (i, 0)),
        pl.BlockSpec((_DIN, _DH), lambda i: (0, 0)),
        pl.BlockSpec((_BN, 2), lambda i: (i, 0)),
    ],
    out_specs=(
        pl.BlockSpec((_BN, 2, _DH // 2), lambda i: (i, 0, 0)),
        pl.BlockSpec((_BN, 1), lambda i: (i, 0)),
    ),
    out_shape=(jax.ShapeDtypeStruct((_NPAD, 2, _DH // 2), _F32),
               jax.ShapeDtypeStruct((_NPAD, 1), _F32)),
)


def _b2_body(agg_ref, gf_ref, dinv_ref, b_ref, w_ref, gf2_ref):
    dinv3 = dinv_ref[...][:, :, None]
    o = jnp.maximum((agg_ref[0] + gf_ref[...]) * dinv3 + b_ref[...], 0.0)
    w = w_ref[...]
    h = (jnp.dot(o[:, 0, :], w[:_DH // 2, :],
                 preferred_element_type=_F32, precision=_HIGH)
         + jnp.dot(o[:, 1, :], w[_DH // 2:, :],
                   preferred_element_type=_F32, precision=_HIGH))
    g = h * dinv_ref[...]
    gf2_ref[:, 0, :] = g[:, :_DH // 2]
    gf2_ref[:, 1, :] = g[:, _DH // 2:]


_b2_call = pl.pallas_call(
    _b2_body,
    grid=(_N // _BN,),
    in_specs=[
        pl.BlockSpec((1, _BN, 2, _DH // 2), _half_map4),
        pl.BlockSpec((_BN, 2, _DH // 2), lambda i: (i, 0, 0)),
        pl.BlockSpec((_BN, 1), lambda i: (i, 0)),
        pl.BlockSpec((1, 2, _DH // 2), lambda i: (0, 0, 0)),
        pl.BlockSpec((_DH, _DH), lambda i: (0, 0)),
    ],
    out_specs=pl.BlockSpec((_BN, 2, _DH // 2), lambda i: (i, 0, 0)),
    out_shape=jax.ShapeDtypeStruct((_NPAD, 2, _DH // 2), _F32),
)


def _b3_body(agg_ref, gf_ref, dinv_ref, b_ref, w_ref, g3_ref):
    dinv3 = dinv_ref[...][:, :, None]
    o = jnp.maximum((agg_ref[0] + gf_ref[...]) * dinv3 + b_ref[...], 0.0)
    w = w_ref[...]
    h = (jnp.dot(o[:, 0, :], w[:_DH // 2, :],
                 preferred_element_type=_F32, precision=_HIGH)
         + jnp.dot(o[:, 1, :], w[_DH // 2:, :],
                   preferred_element_type=_F32, precision=_HIGH))
    g3_ref[...] = h * dinv_ref[...]


_b3_call = pl.pallas_call(
    _b3_body,
    grid=(_N // _BN,),
    in_specs=[
        pl.BlockSpec((1, _BN, 2, _DH // 2), _half_map4),
        pl.BlockSpec((_BN, 2, _DH // 2), lambda i: (i, 0, 0)),
        pl.BlockSpec((_BN, 1), lambda i: (i, 0)),
        pl.BlockSpec((1, 2, _DH // 2), lambda i: (0, 0, 0)),
        pl.BlockSpec((_DH, _DOUT), lambda i: (0, 0)),
    ],
    out_specs=pl.BlockSpec((_BN, _DOUT), lambda i: (i, 0)),
    out_shape=jax.ShapeDtypeStruct((_NPAD, _DOUT), _F32),
)


def _b4_body(p0_ref, p1_ref, g3_ref, dinv_ref, b_ref, out_ref):
    out_ref[...] = (dinv_ref[...] * (p0_ref[...] + p1_ref[...] - g3_ref[...])
                    + b_ref[...])


_b4_call = pl.pallas_call(
    _b4_body,
    grid=(_N // _BNO,),
    in_specs=[
        pl.BlockSpec((_BNO, _DOUT), lambda i: (i, 0)),
        pl.BlockSpec((_BNO, _DOUT), lambda i: (i, 0)),
        pl.BlockSpec((_BNO, _DOUT), lambda i: (i, 0)),
        pl.BlockSpec((_BNO, 1), lambda i: (i, 0)),
        pl.BlockSpec((1, _DOUT), lambda i: (0, 0)),
    ],
    out_specs=pl.BlockSpec((_BNO, _DOUT), lambda i: (i, 0)),
    out_shape=jax.ShapeDtypeStruct((_N, _DOUT), _F32),
)


def kernel(x, edge_index, W1, b1, W2, b2, W3, b3):
    src = edge_index[0]
    dst = edge_index[1]
    srcp = jnp.pad(src, (0, _EPAD - _E))
    dstp = jnp.pad(dst, (0, _EPAD - _E), constant_values=_TRASH)
    src16 = srcp.reshape(_NS, _NB16, _BK)
    dst16 = dstp.reshape(_NS, _NB16, _BK)
    src32 = srcp.reshape(_NC * _NS, _NB32, _BK)
    dst32 = dstp.reshape(_NC * _NS, _NB32, _BK)
    ones = jnp.ones((_BK,), _F32)
    zeros = jnp.zeros((_DEGPAD,), _F32)
    zrows = jnp.zeros((2 * _HROWS, _DH // 2), _F32)

    degp = _deg_call(dst32, ones, zeros)
    degt = degp.T
    posa, posb, dlb = _pos_call(dst16)
    sa, da, sb, db = _scat_call(src16, dst16, posa, posb, dlb)

    def _regions(a, m=1):
        return (a.reshape(_NS, m * _CAPP)[:, :m * _CAP]
                .reshape(_NS, m * _CROWS, _BK))

    sa, sb = _regions(sa), _regions(sb)
    da, db = _regions(da, 2), _regions(db, 2)

    gf, dinv = _b1_call(x, W1, degt)
    agg = _spmm_part_call(gf, sa, da, sb, db, zrows)
    gf2 = _b2_call(agg.reshape(_NC, _HROWS, 2, _DH // 2), gf, dinv,
                   b1.reshape(1, 2, _DH // 2), W2)
    agg = _spmm_part_call(gf2, sa, da, sb, db, zrows)
    g3 = _b3_call(agg.reshape(_NC, _HROWS, 2, _DH // 2), gf2, dinv,
                  b2.reshape(1, 2, _DH // 2), W3)
    p0, p1 = _spmm_edge_call(g3, src32, dst32)
    out = _b4_call(p0, p1, g3, dinv, b3.reshape(1, _DOUT))
    return out


# speedup vs baseline: 2.2123x; 2.2123x over previous
import functools

import jax
import jax.numpy as jnp
from jax import lax
from jax.experimental import pallas as pl
from jax.experimental.pallas import tpu as pltpu
from jax.experimental.pallas import tpu_sc as plsc

_N = 10000
_E = 320000
_DIN, _DH, _DOUT = 128, 256, 128
_NC, _NS, _BK = 2, 16, 128
_TRASH = _N
_NPAD = 10240
_DEGPAD = _NPAD
_DEGSL = _DEGPAD // _NS
_RPT = _NPAD // _NS
_NB16 = 160
_NB32 = 80
_EPAD = _NS * _NB16 * _BK
_CH = 16
_F32 = jnp.float32
_HIGH = lax.Precision.HIGHEST

_MESH = plsc.VectorSubcoreMesh(
    core_axis_name="c", subcore_axis_name="s",
    num_cores=_NC, num_subcores=_NS)


def _deg_body(dst_h, ones_h, zero_h, deg_h, dst_v, ones_v, acc, sem):
    del sem
    c = lax.axis_index("c")
    s = lax.axis_index("s")
    w = c * _NS + s
    pltpu.sync_copy(dst_h.at[w], dst_v)
    pltpu.sync_copy(ones_h, ones_v)
    z0 = s * _DEGSL
    pltpu.sync_copy(zero_h.at[pl.ds(z0, _DEGSL)], acc.at[pl.ds(z0, _DEGSL)])
    plsc.subcore_barrier()

    def body(j, carry):
        pltpu.sync_copy(ones_v, acc.at[dst_v.at[j]], add=True)
        return carry

    lax.fori_loop(0, _NB32, body, 0)
    plsc.subcore_barrier()
    pltpu.sync_copy(acc.at[pl.ds(z0, _DEGSL)], deg_h.at[c, pl.ds(z0, _DEGSL)])


_deg_call = pl.kernel(
    _deg_body,
    out_type=jax.ShapeDtypeStruct((_NC, _DEGPAD), _F32),
    mesh=_MESH,
    scratch_types=[
        pltpu.VMEM((_NB32, _BK), jnp.int32),
        pltpu.VMEM((_BK,), _F32),
        pltpu.VMEM_SHARED((_DEGPAD,), _F32),
        pltpu.SemaphoreType.DMA,
    ],
)


def _spmm_col_body(g0_h, g1_h, src_h, dst_h, a0_h, a1_h,
                   src_v, dst_v, rows0, rows1, acc, sem0, sem1):
    c = lax.axis_index("c")
    s = lax.axis_index("s")
    r0 = s * _RPT

    @pl.when(c == 0)
    def _():
        pltpu.sync_copy(g0_h.at[pl.ds(r0, _RPT)], acc.at[pl.ds(r0, _RPT)])

    @pl.when(c == 1)
    def _():
        pltpu.sync_copy(g1_h.at[pl.ds(r0, _RPT)], acc.at[pl.ds(r0, _RPT)])

    plsc.subcore_barrier()

    def edge_loop(g_h):
        bufs = (rows0, rows1)
        sems = (sem0, sem1)

        def outer(k, carry):
            pltpu.sync_copy(src_h.at[s, pl.ds(k * _CH, _CH)], src_v)
            pltpu.sync_copy(dst_h.at[s, pl.ds(k * _CH, _CH)], dst_v)
            cps = [None, None]
            cps[0] = pltpu.async_copy(g_h.at[src_v.at[0]], bufs[0], sems[0])
            for j in range(_CH):
                p = j & 1
                if j + 1 < _CH:
                    cps[1 - p] = pltpu.async_copy(
                        g_h.at[src_v.at[j + 1]], bufs[1 - p], sems[1 - p])
                cps[p].wait()
                pltpu.sync_copy(bufs[p], acc.at[dst_v.at[j]], add=True)
            return carry
        lax.fori_loop(0, _NB16 // _CH, outer, 0)

    @pl.when(c == 0)
    def _():
        edge_loop(g0_h)

    @pl.when(c == 1)
    def _():
        edge_loop(g1_h)

    plsc.subcore_barrier()

    @pl.when(c == 0)
    def _():
        pltpu.sync_copy(acc.at[pl.ds(r0, _RPT)], a0_h.at[pl.ds(r0, _RPT)])

    @pl.when(c == 1)
    def _():
        pltpu.sync_copy(acc.at[pl.ds(r0, _RPT)], a1_h.at[pl.ds(r0, _RPT)])


_spmm_col_call = pl.kernel(
    _spmm_col_body,
    out_type=(jax.ShapeDtypeStruct((_NPAD, _DH // 2), _F32),
              jax.ShapeDtypeStruct((_NPAD, _DH // 2), _F32)),
    mesh=_MESH,
    scratch_types=[
        pltpu.VMEM((_CH, _BK), jnp.int32),
        pltpu.VMEM((_CH, _BK), jnp.int32),
        pltpu.VMEM((_BK, _DH // 2), _F32),
        pltpu.VMEM((_BK, _DH // 2), _F32),
        pltpu.VMEM_SHARED((_NPAD, _DH // 2), _F32),
        pltpu.SemaphoreType.DMA,
        pltpu.SemaphoreType.DMA,
    ],
)


def _spmm_edge_body(g_h, src_h, dst_h, p0_h, p1_h,
                    src_v, dst_v, rows0, rows1, acc, sem0, sem1):
    c = lax.axis_index("c")
    s = lax.axis_index("s")
    w = c * _NS + s
    r0 = s * _RPT
    pltpu.sync_copy(g_h.at[pl.ds(r0, _RPT)], acc.at[pl.ds(r0, _RPT)])
    plsc.subcore_barrier()

    bufs = (rows0, rows1)
    sems = (sem0, sem1)

    def outer(k, carry):
        pltpu.sync_copy(src_h.at[w, pl.ds(k * _CH, _CH)], src_v)
        pltpu.sync_copy(dst_h.at[w, pl.ds(k * _CH, _CH)], dst_v)
        cps = [None, None]
        cps[0] = pltpu.async_copy(g_h.at[src_v.at[0]], bufs[0], sems[0])
        for j in range(_CH):
            p = j & 1
            if j + 1 < _CH:
                cps[1 - p] = pltpu.async_copy(
                    g_h.at[src_v.at[j + 1]], bufs[1 - p], sems[1 - p])
            cps[p].wait()
            pltpu.sync_copy(bufs[p], acc.at[dst_v.at[j]], add=True)
        return carry

    lax.fori_loop(0, _NB32 // _CH, outer, 0)
    plsc.subcore_barrier()

    @pl.when(c == 0)
    def _():
        pltpu.sync_copy(acc.at[pl.ds(r0, _RPT)], p0_h.at[pl.ds(r0, _RPT)])

    @pl.when(c == 1)
    def _():
        pltpu.sync_copy(acc.at[pl.ds(r0, _RPT)], p1_h.at[pl.ds(r0, _RPT)])


_spmm_edge_call = pl.kernel(
    _spmm_edge_body,
    out_type=(jax.ShapeDtypeStruct((_NPAD, _DOUT), _F32),
              jax.ShapeDtypeStruct((_NPAD, _DOUT), _F32)),
    mesh=_MESH,
    scratch_types=[
        pltpu.VMEM((_CH, _BK), jnp.int32),
        pltpu.VMEM((_CH, _BK), jnp.int32),
        pltpu.VMEM((_BK, _DOUT), _F32),
        pltpu.VMEM((_BK, _DOUT), _F32),
        pltpu.VMEM_SHARED((_NPAD, _DOUT), _F32),
        pltpu.SemaphoreType.DMA,
        pltpu.SemaphoreType.DMA,
    ],
)


_BN = 1280
_BNO = 2000


def _b1_body(x_ref, w1_ref, degt_ref, g0_ref, g1_ref, dinv_ref):
    deg = degt_ref[:, 0:1] + degt_ref[:, 1:2] + 1.0
    dinv = lax.rsqrt(deg)
    h = jnp.dot(x_ref[...], w1_ref[...],
                preferred_element_type=_F32, precision=_HIGH)
    g = h * dinv
    g0_ref[...] = g[:, :_DH // 2]
    g1_ref[...] = g[:, _DH // 2:]
    dinv_ref[...] = dinv


_b1_call = pl.pallas_call(
    _b1_body,
    grid=(_NPAD // _BN,),
    in_specs=[
        pl.BlockSpec((_BN, _DIN), lambda i: (i, 0)),
        pl.BlockSpec((_DIN, _DH), lambda i: (0, 0)),
        pl.BlockSpec((_BN, 2), lambda i: (i, 0)),
    ],
    out_specs=(
        pl.BlockSpec((_BN, _DH // 2), lambda i: (i, 0)),
        pl.BlockSpec((_BN, _DH // 2), lambda i: (i, 0)),
        pl.BlockSpec((_BN, 1), lambda i: (i, 0)),
    ),
    out_shape=(jax.ShapeDtypeStruct((_NPAD, _DH // 2), _F32),
               jax.ShapeDtypeStruct((_NPAD, _DH // 2), _F32),
               jax.ShapeDtypeStruct((_NPAD, 1), _F32)),
)


def _b2_body(a0_ref, a1_ref, dinv_ref, b_ref, w_ref, h0_ref, h1_ref):
    dinv = dinv_ref[...]
    b = b_ref[...]
    o0 = jnp.maximum(a0_ref[...] * dinv + b[:, :_DH // 2], 0.0)
    o1 = jnp.maximum(a1_ref[...] * dinv + b[:, _DH // 2:], 0.0)
    w = w_ref[...]
    h = (jnp.dot(o0, w[:_DH // 2, :], preferred_element_type=_F32,
                 precision=_HIGH)
         + jnp.dot(o1, w[_DH // 2:, :], preferred_element_type=_F32,
                   precision=_HIGH))
    g = h * dinv
    h0_ref[...] = g[:, :_DH // 2]
    h1_ref[...] = g[:, _DH // 2:]


_b2_call = pl.pallas_call(
    _b2_body,
    grid=(_NPAD // _BN,),
    in_specs=[
        pl.BlockSpec((_BN, _DH // 2), lambda i: (i, 0)),
        pl.BlockSpec((_BN, _DH // 2), lambda i: (i, 0)),
        pl.BlockSpec((_BN, 1), lambda i: (i, 0)),
        pl.BlockSpec((1, _DH), lambda i: (0, 0)),
        pl.BlockSpec((_DH, _DH), lambda i: (0, 0)),
    ],
    out_specs=(
        pl.BlockSpec((_BN, _DH // 2), lambda i: (i, 0)),
        pl.BlockSpec((_BN, _DH // 2), lambda i: (i, 0)),
    ),
    out_shape=(jax.ShapeDtypeStruct((_NPAD, _DH // 2), _F32),
               jax.ShapeDtypeStruct((_NPAD, _DH // 2), _F32)),
)


def _b3_body(a0_ref, a1_ref, dinv_ref, b_ref, w_ref, g3_ref):
    dinv = dinv_ref[...]
    b = b_ref[...]
    o0 = jnp.maximum(a0_ref[...] * dinv + b[:, :_DH // 2], 0.0)
    o1 = jnp.maximum(a1_ref[...] * dinv + b[:, _DH // 2:], 0.0)
    w = w_ref[...]
    h = (jnp.dot(o0, w[:_DH // 2, :], preferred_element_type=_F32,
                 precision=_HIGH)
         + jnp.dot(o1, w[_DH // 2:, :], preferred_element_type=_F32,
                   precision=_HIGH))
    g3_ref[...] = h * dinv


_b3_call = pl.pallas_call(
    _b3_body,
    grid=(_NPAD // _BN,),
    in_specs=[
        pl.BlockSpec((_BN, _DH // 2), lambda i: (i, 0)),
        pl.BlockSpec((_BN, _DH // 2), lambda i: (i, 0)),
        pl.BlockSpec((_BN, 1), lambda i: (i, 0)),
        pl.BlockSpec((1, _DH), lambda i: (0, 0)),
        pl.BlockSpec((_DH, _DOUT), lambda i: (0, 0)),
    ],
    out_specs=pl.BlockSpec((_BN, _DOUT), lambda i: (i, 0)),
    out_shape=jax.ShapeDtypeStruct((_NPAD, _DOUT), _F32),
)


def _b4_body(p0_ref, p1_ref, g3_ref, dinv_ref, b_ref, out_ref):
    out_ref[...] = (dinv_ref[...] * (p0_ref[...] + p1_ref[...] - g3_ref[...])
                    + b_ref[...])


_b4_call = pl.pallas_call(
    _b4_body,
    grid=(_N // _BNO,),
    in_specs=[
        pl.BlockSpec((_BNO, _DOUT), lambda i: (i, 0)),
        pl.BlockSpec((_BNO, _DOUT), lambda i: (i, 0)),
        pl.BlockSpec((_BNO, _DOUT), lambda i: (i, 0)),
        pl.BlockSpec((_BNO, 1), lambda i: (i, 0)),
        pl.BlockSpec((1, _DOUT), lambda i: (0, 0)),
    ],
    out_specs=pl.BlockSpec((_BNO, _DOUT), lambda i: (i, 0)),
    out_shape=jax.ShapeDtypeStruct((_N, _DOUT), _F32),
)


def kernel(x, edge_index, W1, b1, W2, b2, W3, b3):
    src = edge_index[0]
    dst = edge_index[1]
    srcp = jnp.pad(src, (0, _EPAD - _E))
    dstp = jnp.pad(dst, (0, _EPAD - _E), constant_values=_TRASH)
    src16 = srcp.reshape(_NS, _NB16, _BK)
    dst16 = dstp.reshape(_NS, _NB16, _BK)
    src32 = srcp.reshape(_NC * _NS, _NB32, _BK)
    dst32 = dstp.reshape(_NC * _NS, _NB32, _BK)
    ones = jnp.ones((_BK,), _F32)
    zeros = jnp.zeros((_DEGPAD,), _F32)

    degp = _deg_call(dst32, ones, zeros)
    degt = degp.T

    g0, g1, dinv = _b1_call(x, W1, degt)
    a0, a1 = _spmm_col_call(g0, g1, src16, dst16)
    h0, h1 = _b2_call(a0, a1, dinv, b1.reshape(1, _DH), W2)
    a0, a1 = _spmm_col_call(h0, h1, src16, dst16)
    g3 = _b3_call(a0, a1, dinv, b2.reshape(1, _DH), W3)
    p0, p1 = _spmm_edge_call(g3, src32, dst32)
    out = _b4_call(p0, p1, g3, dinv, b3.reshape(1, _DOUT))
    return out
